# Initial kernel scaffold; baseline (speedup 1.0000x reference)
import jax
import jax.numpy as jnp
from jax.experimental import pallas as pl


def kernel(X, mask):
    raise NotImplementedError("write your pallas kernel here")


# speedup vs baseline: 1.0443x; 1.0443x over previous
import functools

import jax
import jax.numpy as jnp
from jax.experimental import pallas as pl

_RATIO = 0.1


def _roll_left(x, j):
    return jnp.concatenate([x[:, j:], x[:, :j]], axis=1)


def _topk_sort_body(x_ref, mask_ref, probs_ref, idx_ref, *, N, K):
    v = x_ref[...] * mask_ref[...]
    col = jax.lax.broadcasted_iota(jnp.int32, v.shape, 1)
    idx = col

    k = 2
    while k <= N:
        j = k // 2
        while j >= 1:
            bit0 = (col & j) == 0
            pv = jnp.where(bit0, _roll_left(v, j), _roll_left(v, N - j))
            pi = jnp.where(bit0, _roll_left(idx, j), _roll_left(idx, N - j))
            desc = (col & k) == 0
            keep_first = bit0 == desc
            sgt = (v > pv) | ((v == pv) & (idx < pi))
            take_self = sgt == keep_first
            v = jnp.where(take_self, v, pv)
            idx = jnp.where(take_self, idx, pi)
            j //= 2
        k *= 2

    msum = jnp.sum(mask_ref[...], axis=1, keepdims=True)
    ks = jnp.maximum(jnp.floor(_RATIO * msum), 1.0)
    vsum = jnp.sum(jnp.where(col < K, v, 0.0), axis=1, keepdims=True)
    probs_ref[...] = vsum / ks
    idx_ref[...] = idx[:, :K]


def _topk_call(X, mask, interpret=False):
    B, N = X.shape
    K = max(int(_RATIO * N), 1)
    return pl.pallas_call(
        functools.partial(_topk_sort_body, N=N, K=K),
        out_shape=(
            jax.ShapeDtypeStruct((B, 1), jnp.float32),
            jax.ShapeDtypeStruct((B, K), jnp.int32),
        ),
        interpret=interpret,
    )(X, mask)


def kernel(X, mask):
    return _topk_call(X, mask)


# speedup vs baseline: 3.4158x; 3.2709x over previous
import functools

import jax
import jax.numpy as jnp
from jax import lax
from jax.experimental import pallas as pl
from jax.experimental.pallas import tpu as pltpu
from jax.experimental.pallas import tpu_sc as plsc

_RATIO = 0.1
_ONE_BITS = 0x3F800000
_CAP = 1024
_PAD = 1040


def _roll_left(x, j):
    return jnp.concatenate([x[:, j:], x[:, :j]], axis=1)


def _bitonic_pairs(v, idx, col, N):
    k = 2
    while k <= N:
        j = k // 2
        while j >= 1:
            bit0 = (col & j) == 0
            pv = jnp.where(bit0, _roll_left(v, j), _roll_left(v, N - j))
            pi = jnp.where(bit0, _roll_left(idx, j), _roll_left(idx, N - j))
            desc = (col & k) == 0
            keep_first = bit0 == desc
            sgt = (v > pv) | ((v == pv) & (idx < pi))
            take_self = sgt == keep_first
            v = jnp.where(take_self, v, pv)
            idx = jnp.where(take_self, idx, pi)
            j //= 2
        k *= 2
    return v, idx


def _thresh_body(x_ref, mask_ref, v_ref, dst_ref, probs_ref, *, B, N, K):
    v = x_ref[...] * mask_ref[...]
    v_ref[...] = v
    bits = lax.bitcast_convert_type(v, jnp.int32)
    kf = jnp.float32(K)

    def it(_, carry):
        lo, hi = carry
        mid = (lo + hi) >> 1
        c = jnp.sum((bits >= mid).astype(jnp.float32), axis=1, keepdims=True)
        ge = c >= kf
        return jnp.where(ge, mid, lo), jnp.where(ge, hi, mid)

    lo0 = jnp.zeros((B, 1), jnp.int32)
    hi0 = jnp.full((B, 1), _ONE_BITS, jnp.int32)
    lo, _ = lax.fori_loop(0, 30, it, (lo0, hi0))
    thr = lax.bitcast_convert_type(lo, jnp.float32)
    gt = bits > lo
    cgt = jnp.sum(gt.astype(jnp.float32), axis=1, keepdims=True)
    sgt = jnp.sum(jnp.where(gt, v, 0.0), axis=1, keepdims=True)
    msum = jnp.sum(mask_ref[...], axis=1, keepdims=True)
    ks = jnp.maximum(jnp.floor(_RATIO * msum), 1.0)
    probs_ref[...] = (sgt + thr * (kf - cgt)) / ks

    sel = (bits >= lo).astype(jnp.int32)
    cum = sel
    d = 1
    while d < N:
        shifted = jnp.concatenate(
            [jnp.zeros((B, d), jnp.int32), cum[:, : N - d]], axis=1)
        cum = cum + shifted
        d *= 2
    excl = cum - sel
    col = lax.broadcasted_iota(jnp.int32, (B, N), 1)
    trash = _CAP + (col & 15)
    dst_ref[...] = jnp.where((sel == 1) & (excl < _CAP), excl, trash)


def _compact_body(v_hbm, dst_hbm, ovals_hbm, oidx_hbm,
                  vrow, dstrow, ovals_l, oidx_l, *, N, rows_per_tile):
    nc = 2
    wid = lax.axis_index("s") * nc + lax.axis_index("c")
    lane = lax.broadcasted_iota(jnp.int32, (16,), 0)

    def init_body(i, _):
        ovals_l[pl.ds(i * 16, 16)] = jnp.full((16,), -1.0, jnp.float32)
        oidx_l[pl.ds(i * 16, 16)] = jnp.full((16,), 1 << 20, jnp.int32)
        return 0

    def step(i, _):
        vec = vrow[pl.ds(i * 16, 16)]
        dst = dstrow[pl.ds(i * 16, 16)]
        plsc.store_scatter(ovals_l, [dst], vec)
        plsc.store_scatter(oidx_l, [dst], lane + i * 16)
        return 0

    for r in range(rows_per_tile):
        row = wid * rows_per_tile + r
        pltpu.sync_copy(v_hbm.at[row], vrow)
        pltpu.sync_copy(dst_hbm.at[row], dstrow)
        lax.fori_loop(0, _PAD // 16, init_body, 0)
        lax.fori_loop(0, N // 16, step, 0)
        pltpu.sync_copy(ovals_l, ovals_hbm.at[row])
        pltpu.sync_copy(oidx_l, oidx_hbm.at[row])


def _sort_body(cv_ref, ci_ref, idx_ref, *, B, K):
    v = cv_ref[...]
    idx = ci_ref[...]
    col = lax.broadcasted_iota(jnp.int32, (B, _CAP), 1)
    v, idx = _bitonic_pairs(v, idx, col, _CAP)
    idx_ref[...] = idx[:, :K]


def kernel(X, mask):
    B, N = X.shape
    K = max(int(_RATIO * N), 1)

    v, dst, probs = pl.pallas_call(
        functools.partial(_thresh_body, B=B, N=N, K=K),
        out_shape=(
            jax.ShapeDtypeStruct((B, N), jnp.float32),
            jax.ShapeDtypeStruct((B, N), jnp.int32),
            jax.ShapeDtypeStruct((B, 1), jnp.float32),
        ),
    )(X, mask)

    mesh = plsc.VectorSubcoreMesh(core_axis_name="c", subcore_axis_name="s")
    compact = functools.partial(
        pl.kernel,
        mesh=mesh,
        out_type=(
            jax.ShapeDtypeStruct((B, _PAD), jnp.float32),
            jax.ShapeDtypeStruct((B, _PAD), jnp.int32),
        ),
        scratch_types=[
            pltpu.VMEM((N,), jnp.float32),
            pltpu.VMEM((N,), jnp.int32),
            pltpu.VMEM((_PAD,), jnp.float32),
            pltpu.VMEM((_PAD,), jnp.int32),
        ],
        compiler_params=pltpu.CompilerParams(needs_layout_passes=False),
    )(functools.partial(_compact_body, N=N, rows_per_tile=B // 32))
    cvals, cidx = compact(v, dst)

    idx = pl.pallas_call(
        functools.partial(_sort_body, B=B, K=K),
        out_shape=jax.ShapeDtypeStruct((B, K), jnp.int32),
    )(cvals[:, :_CAP], cidx[:, :_CAP])

    return probs, idx
